# baseline (speedup 1.0000x reference)
import functools

import jax
import jax.numpy as jnp
from jax import lax
from jax.experimental import pallas as pl
from jax.experimental.pallas import tpu as pltpu
from jax.experimental.pallas import tpu_sc as plsc

N = 10000
E = 320000
D = 128

NC = 2
NS = 16
NW = NC * NS
EPW = E // NW
K = 80
NCHUNK = EPW // K
KA = 40
NCA = EPW // KA
RPT = 624
TAIL = N - NS * RPT
RZ = 48

_sc_mesh = plsc.VectorSubcoreMesh(core_axis_name="c", subcore_axis_name="s")
_sc_params = pltpu.CompilerParams(needs_layout_passes=False,
                                  use_tc_tiling_on_sc=False)


@functools.partial(
    pl.kernel,
    out_type=jax.ShapeDtypeStruct((NC * N, 16), jnp.float32),
    mesh=_sc_mesh,
    compiler_params=_sc_params,
    scratch_types=[
        pltpu.VMEM_SHARED((N, 16), jnp.float32),
        pltpu.VMEM((NCHUNK, K), jnp.int32),
        pltpu.VMEM((EPW,), jnp.float32),
        pltpu.VMEM((K, 16), jnp.float32),
        pltpu.VMEM((RZ, 16), jnp.float32),
    ],
)
def _deg_kernel(dst_hbm, ew_hbm, out_hbm, acc_sh, didx_v, ew_v, rows_v, zbuf_v):
    cid = lax.axis_index("c")
    sid = lax.axis_index("s")
    wid = cid * NS + sid

    def zrow(i, carry):
        zbuf_v[i, :] = jnp.zeros((16,), jnp.float32)
        return carry

    lax.fori_loop(0, RZ, zrow, 0)
    for r in range(RPT // RZ):
        pltpu.sync_copy(zbuf_v, acc_sh.at[pl.ds(sid * RPT + r * RZ, RZ)])

    @pl.when(sid == NS - 1)
    def _():
        pltpu.sync_copy(zbuf_v.at[pl.ds(0, TAIL)],
                        acc_sh.at[pl.ds(NS * RPT, TAIL)])

    plsc.subcore_barrier()

    pltpu.sync_copy(dst_hbm.at[wid], didx_v)
    pltpu.sync_copy(ew_hbm.at[wid], ew_v)

    def chunk(c, carry):
        def fill(e, carry2):
            w16 = plsc.load_gather(ew_v, [jnp.full((16,), c * K + e, jnp.int32)])
            rows_v[e, :] = w16
            return carry2

        lax.fori_loop(0, K, fill, 0)
        pltpu.sync_copy(rows_v, acc_sh.at[didx_v.at[c]], add=True)
        return carry

    lax.fori_loop(0, NCHUNK, chunk, 0)
    plsc.subcore_barrier()
    pltpu.sync_copy(
        acc_sh.at[pl.ds(sid * RPT, RPT)],
        out_hbm.at[pl.ds(cid * N + sid * RPT, RPT)],
    )

    @pl.when(sid == NS - 1)
    def _():
        pltpu.sync_copy(acc_sh.at[pl.ds(NS * RPT, TAIL)],
                        out_hbm.at[pl.ds(cid * N + NS * RPT, TAIL)])


@functools.partial(
    pl.kernel,
    out_type=jax.ShapeDtypeStruct((E,), jnp.float32),
    mesh=_sc_mesh,
    compiler_params=_sc_params,
    scratch_types=[
        pltpu.VMEM((N,), jnp.float32),
        pltpu.VMEM((EPW,), jnp.int32),
        pltpu.VMEM((EPW,), jnp.float32),
        pltpu.VMEM((EPW,), jnp.float32),
    ],
)
def _edgew_kernel(dinv_hbm, src_hbm, ew_hbm, w2_hbm, dinv_v, src_v, ew_v, w2_v):
    cid = lax.axis_index("c")
    sid = lax.axis_index("s")
    wid = cid * NS + sid
    pltpu.sync_copy(dinv_hbm, dinv_v)
    pltpu.sync_copy(src_hbm.at[wid], src_v)
    pltpu.sync_copy(ew_hbm.at[wid], ew_v)

    def step(j, carry):
        sl = pl.ds(j * 16, 16)
        g = plsc.load_gather(dinv_v, [src_v[sl]])
        w2_v[sl] = ew_v[sl] * g
        return carry

    lax.fori_loop(0, EPW // 16, step, 0)
    pltpu.sync_copy(w2_v, w2_hbm.at[pl.ds(wid * EPW, EPW)])


@functools.partial(
    pl.kernel,
    out_type=jax.ShapeDtypeStruct((NC * N, D), jnp.float32),
    mesh=_sc_mesh,
    compiler_params=_sc_params,
    scratch_types=[
        pltpu.VMEM_SHARED((N, D), jnp.float32),
        pltpu.VMEM((NCA, KA), jnp.int32),
        pltpu.VMEM((NCA, KA), jnp.int32),
        pltpu.VMEM((EPW,), jnp.float32),
        pltpu.VMEM((KA, D), jnp.bfloat16),
        pltpu.VMEM((KA, D), jnp.bfloat16),
        pltpu.VMEM((KA, D), jnp.float32),
        pltpu.VMEM((KA, D), jnp.float32),
        pltpu.SemaphoreType.DMA,
        pltpu.SemaphoreType.DMA,
        pltpu.SemaphoreType.DMA,
        pltpu.SemaphoreType.DMA,
    ],
)
def _agg_kernel(hbf_hbm, src_hbm, dst_hbm, w2_hbm, out_hbm,
                acc_sh, sidx_v, didx_v, w2_v, bf_a, bf_b, f_a, f_b,
                gsem_a, gsem_b, ssem_a, ssem_b):
    cid = lax.axis_index("c")
    sid = lax.axis_index("s")
    wid = cid * NS + sid

    def zrow(i, carry):
        for j in range(D // 16):
            f_a[i, pl.ds(j * 16, 16)] = jnp.zeros((16,), jnp.float32)
        return carry

    lax.fori_loop(0, KA, zrow, 0)
    for r in range(RPT // KA):
        pltpu.sync_copy(f_a, acc_sh.at[pl.ds(sid * RPT + r * KA, KA)])
    pltpu.sync_copy(f_a.at[pl.ds(0, RPT - (RPT // KA) * KA)],
                    acc_sh.at[pl.ds(sid * RPT + (RPT // KA) * KA,
                                    RPT - (RPT // KA) * KA)])

    @pl.when(sid == NS - 1)
    def _():
        pltpu.sync_copy(f_a.at[pl.ds(0, TAIL)],
                        acc_sh.at[pl.ds(NS * RPT, TAIL)])

    plsc.subcore_barrier()

    pltpu.sync_copy(src_hbm.at[wid], sidx_v)
    pltpu.sync_copy(dst_hbm.at[wid], didx_v)
    pltpu.sync_copy(w2_hbm.at[wid], w2_v)

    def scale(src_bf, dst_f, c):
        def scale1(e, carry2):
            w16 = plsc.load_gather(
                w2_v, [jnp.full((16,), c * KA + e, jnp.int32)])
            for j in range(D // 32):
                v = src_bf[e, pl.ds(32 * j, 32)]
                lo, hi = plsc.unpack(v, format=plsc.PackFormat.INTERLEAVED)
                dst_f[e, pl.ds(32 * j, 16)] = lo * w16
                dst_f[e, pl.ds(32 * j + 16, 16)] = hi * w16
            return carry2

        lax.fori_loop(0, KA, scale1, 0)

    pltpu.async_copy(hbf_hbm.at[sidx_v.at[0]], bf_a, gsem_a)
    pltpu.async_copy(hbf_hbm.at[sidx_v.at[1]], bf_b, gsem_b)

    def phase(i, c, bf, f, gsem, ssem):
        pltpu.make_async_copy(hbf_hbm.at[sidx_v.at[c]], bf, gsem).wait()

        @pl.when(i > 0)
        def _():
            pltpu.make_async_copy(f, acc_sh.at[didx_v.at[c]], ssem).wait()

        scale(bf, f, c)
        pltpu.async_copy(f, acc_sh.at[didx_v.at[c]], ssem, add=True)
        pltpu.async_copy(hbf_hbm.at[sidx_v.at[jnp.minimum(c + 2, NCA - 1)]],
                         bf, gsem)

    def body(i, carry):
        phase(i, 2 * i, bf_a, f_a, gsem_a, ssem_a)
        phase(i, 2 * i + 1, bf_b, f_b, gsem_b, ssem_b)
        return carry

    lax.fori_loop(0, NCA // 2, body, 0)

    last = NCA - 1
    pltpu.make_async_copy(hbf_hbm.at[sidx_v.at[last]], bf_a, gsem_a).wait()
    pltpu.make_async_copy(hbf_hbm.at[sidx_v.at[last]], bf_b, gsem_b).wait()
    pltpu.make_async_copy(f_a, acc_sh.at[didx_v.at[last]], ssem_a).wait()
    pltpu.make_async_copy(f_b, acc_sh.at[didx_v.at[last]], ssem_b).wait()
    plsc.subcore_barrier()
    pltpu.sync_copy(
        acc_sh.at[pl.ds(sid * RPT, RPT)],
        out_hbm.at[pl.ds(cid * N + sid * RPT, RPT)],
    )

    @pl.when(sid == NS - 1)
    def _():
        pltpu.sync_copy(acc_sh.at[pl.ds(NS * RPT, TAIL)],
                        out_hbm.at[pl.ds(cid * N + NS * RPT, TAIL)])


BM = 1000


def _bf16_interleaved(h):
    h4 = h.reshape(h.shape[0], D // 32, 2, 16)
    return jnp.swapaxes(h4, 2, 3).reshape(h.shape[0], D).astype(jnp.bfloat16)


@functools.partial(
    pl.pallas_call,
    grid=(N // BM,),
    in_specs=[
        pl.BlockSpec((BM, D), lambda i: (i, 0)),
        pl.BlockSpec((D, D), lambda i: (0, 0)),
        pl.BlockSpec((BM, 16), lambda i: (i, 0)),
        pl.BlockSpec((BM, 16), lambda i: (i, 0)),
    ],
    out_specs=[
        pl.BlockSpec((BM, D), lambda i: (i, 0)),
        pl.BlockSpec((BM, D), lambda i: (i, 0)),
        pl.BlockSpec((BM, 16), lambda i: (i, 0)),
    ],
    out_shape=[
        jax.ShapeDtypeStruct((N, D), jnp.float32),
        jax.ShapeDtypeStruct((N, D), jnp.bfloat16),
        jax.ShapeDtypeStruct((N, 16), jnp.float32),
    ],
)
def _mm_dinv_kernel(x_ref, w_ref, d0_ref, d1_ref, h_ref, hbf_ref, dinv_ref):
    h = jnp.dot(x_ref[...], w_ref[...], preferred_element_type=jnp.float32)
    h_ref[...] = h
    hbf_ref[...] = _bf16_interleaved(h)
    deg = d0_ref[...] + d1_ref[...] + 1.0
    dinv_ref[...] = jnp.where(deg > 0, lax.rsqrt(deg), 0.0)


@functools.partial(
    pl.pallas_call,
    grid=(N // BM,),
    in_specs=[
        pl.BlockSpec((BM, D), lambda i: (i, 0)),
        pl.BlockSpec((BM, D), lambda i: (i, 0)),
        pl.BlockSpec((BM, D), lambda i: (i, 0)),
        pl.BlockSpec((BM, 16), lambda i: (i, 0)),
        pl.BlockSpec((1, D), lambda i: (0, 0)),
        pl.BlockSpec((D, D), lambda i: (0, 0)),
    ],
    out_specs=[
        pl.BlockSpec((BM, D), lambda i: (i, 0)),
        pl.BlockSpec((BM, D), lambda i: (i, 0)),
    ],
    out_shape=[
        jax.ShapeDtypeStruct((N, D), jnp.float32),
        jax.ShapeDtypeStruct((N, D), jnp.bfloat16),
    ],
)
def _combine_mm_kernel(p0_ref, p1_ref, h_ref, dinv_ref, b_ref, w_ref,
                       out_ref, outbf_ref):
    d1 = dinv_ref[:, :1]
    y = jnp.maximum(
        d1 * (p0_ref[...] + p1_ref[...]) + (d1 * d1) * h_ref[...] + b_ref[...],
        0.0,
    )
    h = jnp.dot(y, w_ref[...], preferred_element_type=jnp.float32)
    out_ref[...] = h
    outbf_ref[...] = _bf16_interleaved(h)


@functools.partial(
    pl.pallas_call,
    grid=(N // BM,),
    in_specs=[
        pl.BlockSpec((BM, D), lambda i: (i, 0)),
        pl.BlockSpec((BM, D), lambda i: (i, 0)),
        pl.BlockSpec((BM, D), lambda i: (i, 0)),
        pl.BlockSpec((BM, 16), lambda i: (i, 0)),
        pl.BlockSpec((1, D), lambda i: (0, 0)),
    ],
    out_specs=pl.BlockSpec((BM, D), lambda i: (i, 0)),
    out_shape=jax.ShapeDtypeStruct((N, D), jnp.float32),
)
def _combine_kernel(p0_ref, p1_ref, h_ref, dinv_ref, b_ref, out_ref):
    d1 = dinv_ref[:, :1]
    out_ref[...] = jnp.maximum(
        d1 * (p0_ref[...] + p1_ref[...]) + (d1 * d1) * h_ref[...] + b_ref[...],
        0.0,
    )


def kernel(x, edge_index, edge_attr, W0, b0, W1, b1, W2, b2):
    src = edge_index[0].astype(jnp.int32)
    dst = edge_index[1].astype(jnp.int32)
    src3 = src.reshape(NW, NCA, KA)
    dst3 = dst.reshape(NW, NCA, KA)
    dst3d = dst.reshape(NW, NCHUNK, K)
    src2 = src.reshape(NW, EPW)
    ew2 = edge_attr.astype(jnp.float32).reshape(NW, EPW)

    degp = _deg_kernel(dst3d, ew2)
    h0, h0bf, dinv16 = _mm_dinv_kernel(x, W0, degp[:N], degp[N:])
    dinv1 = dinv16[:, 0]
    w2 = _edgew_kernel(dinv1, src2, ew2).reshape(NW, EPW)

    b0r = b0.reshape(1, D)
    b1r = b1.reshape(1, D)
    b2r = b2.reshape(1, D)

    P = _agg_kernel(h0bf, src3, dst3, w2)
    h1, h1bf = _combine_mm_kernel(P[:N], P[N:], h0, dinv16, b0r, W1)
    P = _agg_kernel(h1bf, src3, dst3, w2)
    h2, h2bf = _combine_mm_kernel(P[:N], P[N:], h1, dinv16, b1r, W2)
    P = _agg_kernel(h2bf, src3, dst3, w2)
    return _combine_kernel(P[:N], P[N:], h2, dinv16, b2r)


# speedup vs baseline: 2.0502x; 2.0502x over previous
import functools

import jax
import jax.numpy as jnp
from jax import lax
from jax.experimental import pallas as pl
from jax.experimental.pallas import tpu as pltpu
from jax.experimental.pallas import tpu_sc as plsc

N = 10000
E = 320000
D = 128

NC = 2
NS = 16
NW = NC * NS
EPW = E // NW
K = 80
NCHUNK = EPW // K
KA = 40
NCA = EPW // KA
RPT = 624
TAIL = N - NS * RPT
RZ = 48

_sc_mesh = plsc.VectorSubcoreMesh(core_axis_name="c", subcore_axis_name="s")
_sc_params = pltpu.CompilerParams(needs_layout_passes=False,
                                  use_tc_tiling_on_sc=False)


@functools.partial(
    pl.kernel,
    out_type=jax.ShapeDtypeStruct((NC * N, 16), jnp.float32),
    mesh=_sc_mesh,
    compiler_params=_sc_params,
    scratch_types=[
        pltpu.VMEM_SHARED((N, 16), jnp.float32),
        pltpu.VMEM((NCHUNK, K), jnp.int32),
        pltpu.VMEM((EPW,), jnp.float32),
        pltpu.VMEM((K, 16), jnp.float32),
        pltpu.VMEM((RZ, 16), jnp.float32),
    ],
)
def _deg_kernel(dst_hbm, ew_hbm, out_hbm, acc_sh, didx_v, ew_v, rows_v, zbuf_v):
    cid = lax.axis_index("c")
    sid = lax.axis_index("s")
    wid = cid * NS + sid

    def zrow(i, carry):
        zbuf_v[i, :] = jnp.zeros((16,), jnp.float32)
        return carry

    lax.fori_loop(0, RZ, zrow, 0)
    for r in range(RPT // RZ):
        pltpu.sync_copy(zbuf_v, acc_sh.at[pl.ds(sid * RPT + r * RZ, RZ)])

    @pl.when(sid == NS - 1)
    def _():
        pltpu.sync_copy(zbuf_v.at[pl.ds(0, TAIL)],
                        acc_sh.at[pl.ds(NS * RPT, TAIL)])

    plsc.subcore_barrier()

    pltpu.sync_copy(dst_hbm.at[wid], didx_v)
    pltpu.sync_copy(ew_hbm.at[wid], ew_v)

    def chunk(c, carry):
        def fill(e, carry2):
            w16 = plsc.load_gather(ew_v, [jnp.full((16,), c * K + e, jnp.int32)])
            rows_v[e, :] = w16
            return carry2

        lax.fori_loop(0, K, fill, 0)
        pltpu.sync_copy(rows_v, acc_sh.at[didx_v.at[c]], add=True)
        return carry

    lax.fori_loop(0, NCHUNK, chunk, 0)
    plsc.subcore_barrier()
    pltpu.sync_copy(
        acc_sh.at[pl.ds(sid * RPT, RPT)],
        out_hbm.at[pl.ds(cid * N + sid * RPT, RPT)],
    )

    @pl.when(sid == NS - 1)
    def _():
        pltpu.sync_copy(acc_sh.at[pl.ds(NS * RPT, TAIL)],
                        out_hbm.at[pl.ds(cid * N + NS * RPT, TAIL)])


@functools.partial(
    pl.kernel,
    out_type=jax.ShapeDtypeStruct((E,), jnp.float32),
    mesh=_sc_mesh,
    compiler_params=_sc_params,
    scratch_types=[
        pltpu.VMEM((N,), jnp.float32),
        pltpu.VMEM((EPW,), jnp.int32),
        pltpu.VMEM((EPW,), jnp.float32),
        pltpu.VMEM((EPW,), jnp.float32),
    ],
)
def _edgew_kernel(dinv_hbm, src_hbm, ew_hbm, w2_hbm, dinv_v, src_v, ew_v, w2_v):
    cid = lax.axis_index("c")
    sid = lax.axis_index("s")
    wid = cid * NS + sid
    pltpu.sync_copy(dinv_hbm, dinv_v)
    pltpu.sync_copy(src_hbm.at[wid], src_v)
    pltpu.sync_copy(ew_hbm.at[wid], ew_v)

    def step(j, carry):
        sl = pl.ds(j * 16, 16)
        g = plsc.load_gather(dinv_v, [src_v[sl]])
        w2_v[sl] = ew_v[sl] * g
        return carry

    lax.fori_loop(0, EPW // 16, step, 0)
    pltpu.sync_copy(w2_v, w2_hbm.at[pl.ds(wid * EPW, EPW)])


@functools.partial(
    pl.kernel,
    out_type=jax.ShapeDtypeStruct((NC * N, D), jnp.float32),
    mesh=_sc_mesh,
    compiler_params=_sc_params,
    scratch_types=[
        pltpu.VMEM_SHARED((N, D), jnp.float32),
        pltpu.VMEM((NCA, KA), jnp.int32),
        pltpu.VMEM((NCA, KA), jnp.int32),
        pltpu.VMEM((EPW,), jnp.float32),
        pltpu.VMEM((KA, D), jnp.float32),
        pltpu.VMEM((KA, D), jnp.float32),
        pltpu.VMEM((KA, D), jnp.float32),
        pltpu.VMEM((KA, D), jnp.float32),
        pltpu.SemaphoreType.DMA,
        pltpu.SemaphoreType.DMA,
        pltpu.SemaphoreType.DMA,
        pltpu.SemaphoreType.DMA,
    ],
)
def _agg_kernel(h_hbm, src_hbm, dst_hbm, w2_hbm, out_hbm,
                acc_sh, sidx_v, didx_v, w2_v, g_a, g_b, f_a, f_b,
                gsem_a, gsem_b, ssem_a, ssem_b):
    cid = lax.axis_index("c")
    sid = lax.axis_index("s")
    wid = cid * NS + sid

    def zrow(i, carry):
        for j in range(D // 16):
            f_a[i, pl.ds(j * 16, 16)] = jnp.zeros((16,), jnp.float32)
        return carry

    lax.fori_loop(0, KA, zrow, 0)
    for r in range(RPT // KA):
        pltpu.sync_copy(f_a, acc_sh.at[pl.ds(sid * RPT + r * KA, KA)])
    pltpu.sync_copy(f_a.at[pl.ds(0, RPT - (RPT // KA) * KA)],
                    acc_sh.at[pl.ds(sid * RPT + (RPT // KA) * KA,
                                    RPT - (RPT // KA) * KA)])

    @pl.when(sid == NS - 1)
    def _():
        pltpu.sync_copy(f_a.at[pl.ds(0, TAIL)],
                        acc_sh.at[pl.ds(NS * RPT, TAIL)])

    plsc.subcore_barrier()

    pltpu.sync_copy(src_hbm.at[wid], sidx_v)
    pltpu.sync_copy(dst_hbm.at[wid], didx_v)
    pltpu.sync_copy(w2_hbm.at[wid], w2_v)

    def scale(src_g, dst_f, c):
        def scale1(e, carry2):
            w16 = plsc.load_gather(
                w2_v, [jnp.full((16,), c * KA + e, jnp.int32)])
            for j in range(D // 16):
                sl = pl.ds(j * 16, 16)
                dst_f[e, sl] = src_g[e, sl] * w16
            return carry2

        lax.fori_loop(0, KA, scale1, 0)

    pltpu.async_copy(h_hbm.at[sidx_v.at[0]], g_a, gsem_a)
    pltpu.async_copy(h_hbm.at[sidx_v.at[1]], g_b, gsem_b)

    def phase(i, c, g, f, gsem, ssem):
        pltpu.make_async_copy(h_hbm.at[sidx_v.at[c]], g, gsem).wait()

        @pl.when(i > 0)
        def _():
            pltpu.make_async_copy(f, acc_sh.at[didx_v.at[c]], ssem).wait()

        scale(g, f, c)
        pltpu.async_copy(f, acc_sh.at[didx_v.at[c]], ssem, add=True)
        pltpu.async_copy(h_hbm.at[sidx_v.at[jnp.minimum(c + 2, NCA - 1)]],
                         g, gsem)

    def body(i, carry):
        phase(i, 2 * i, g_a, f_a, gsem_a, ssem_a)
        phase(i, 2 * i + 1, g_b, f_b, gsem_b, ssem_b)
        return carry

    lax.fori_loop(0, NCA // 2, body, 0)

    last = NCA - 1
    pltpu.make_async_copy(h_hbm.at[sidx_v.at[last]], g_a, gsem_a).wait()
    pltpu.make_async_copy(h_hbm.at[sidx_v.at[last]], g_b, gsem_b).wait()
    pltpu.make_async_copy(f_a, acc_sh.at[didx_v.at[last]], ssem_a).wait()
    pltpu.make_async_copy(f_b, acc_sh.at[didx_v.at[last]], ssem_b).wait()
    plsc.subcore_barrier()
    pltpu.sync_copy(
        acc_sh.at[pl.ds(sid * RPT, RPT)],
        out_hbm.at[pl.ds(cid * N + sid * RPT, RPT)],
    )

    @pl.when(sid == NS - 1)
    def _():
        pltpu.sync_copy(acc_sh.at[pl.ds(NS * RPT, TAIL)],
                        out_hbm.at[pl.ds(cid * N + NS * RPT, TAIL)])


BM = 1000


@functools.partial(
    pl.pallas_call,
    grid=(N // BM,),
    in_specs=[
        pl.BlockSpec((BM, D), lambda i: (i, 0)),
        pl.BlockSpec((D, D), lambda i: (0, 0)),
        pl.BlockSpec((BM, 16), lambda i: (i, 0)),
        pl.BlockSpec((BM, 16), lambda i: (i, 0)),
    ],
    out_specs=[
        pl.BlockSpec((BM, D), lambda i: (i, 0)),
        pl.BlockSpec((BM, 16), lambda i: (i, 0)),
    ],
    out_shape=[
        jax.ShapeDtypeStruct((N, D), jnp.float32),
        jax.ShapeDtypeStruct((N, 16), jnp.float32),
    ],
)
def _mm_dinv_kernel(x_ref, w_ref, d0_ref, d1_ref, h_ref, dinv_ref):
    h_ref[...] = jnp.dot(x_ref[...], w_ref[...],
                         preferred_element_type=jnp.float32)
    deg = d0_ref[...] + d1_ref[...] + 1.0
    dinv_ref[...] = jnp.where(deg > 0, lax.rsqrt(deg), 0.0)


@functools.partial(
    pl.pallas_call,
    grid=(N // BM,),
    in_specs=[
        pl.BlockSpec((BM, D), lambda i: (i, 0)),
        pl.BlockSpec((BM, D), lambda i: (i, 0)),
        pl.BlockSpec((BM, D), lambda i: (i, 0)),
        pl.BlockSpec((BM, 16), lambda i: (i, 0)),
        pl.BlockSpec((1, D), lambda i: (0, 0)),
        pl.BlockSpec((D, D), lambda i: (0, 0)),
    ],
    out_specs=pl.BlockSpec((BM, D), lambda i: (i, 0)),
    out_shape=jax.ShapeDtypeStruct((N, D), jnp.float32),
)
def _combine_mm_kernel(p0_ref, p1_ref, h_ref, dinv_ref, b_ref, w_ref, out_ref):
    d1 = dinv_ref[:, :1]
    y = jnp.maximum(
        d1 * (p0_ref[...] + p1_ref[...]) + (d1 * d1) * h_ref[...] + b_ref[...],
        0.0,
    )
    out_ref[...] = jnp.dot(y, w_ref[...], preferred_element_type=jnp.float32)


@functools.partial(
    pl.pallas_call,
    grid=(N // BM,),
    in_specs=[
        pl.BlockSpec((BM, D), lambda i: (i, 0)),
        pl.BlockSpec((BM, D), lambda i: (i, 0)),
        pl.BlockSpec((BM, D), lambda i: (i, 0)),
        pl.BlockSpec((BM, 16), lambda i: (i, 0)),
        pl.BlockSpec((1, D), lambda i: (0, 0)),
    ],
    out_specs=pl.BlockSpec((BM, D), lambda i: (i, 0)),
    out_shape=jax.ShapeDtypeStruct((N, D), jnp.float32),
)
def _combine_kernel(p0_ref, p1_ref, h_ref, dinv_ref, b_ref, out_ref):
    d1 = dinv_ref[:, :1]
    out_ref[...] = jnp.maximum(
        d1 * (p0_ref[...] + p1_ref[...]) + (d1 * d1) * h_ref[...] + b_ref[...],
        0.0,
    )


def kernel(x, edge_index, edge_attr, W0, b0, W1, b1, W2, b2):
    src = edge_index[0].astype(jnp.int32)
    dst = edge_index[1].astype(jnp.int32)
    src3 = src.reshape(NW, NCA, KA)
    dst3 = dst.reshape(NW, NCA, KA)
    dst3d = dst.reshape(NW, NCHUNK, K)
    src2 = src.reshape(NW, EPW)
    ew2 = edge_attr.astype(jnp.float32).reshape(NW, EPW)

    degp = _deg_kernel(dst3d, ew2)
    h0, dinv16 = _mm_dinv_kernel(x, W0, degp[:N], degp[N:])
    dinv1 = dinv16[:, 0]
    w2 = _edgew_kernel(dinv1, src2, ew2).reshape(NW, EPW)

    b0r = b0.reshape(1, D)
    b1r = b1.reshape(1, D)
    b2r = b2.reshape(1, D)

    P = _agg_kernel(h0, src3, dst3, w2)
    h1 = _combine_mm_kernel(P[:N], P[N:], h0, dinv16, b0r, W1)
    P = _agg_kernel(h1, src3, dst3, w2)
    h2 = _combine_mm_kernel(P[:N], P[N:], h1, dinv16, b1r, W2)
    P = _agg_kernel(h2, src3, dst3, w2)
    return _combine_kernel(P[:N], P[N:], h2, dinv16, b2r)


# speedup vs baseline: 2.2120x; 1.0789x over previous
import functools

import jax
import jax.numpy as jnp
from jax import lax
from jax.experimental import pallas as pl
from jax.experimental.pallas import tpu as pltpu
from jax.experimental.pallas import tpu_sc as plsc

N = 10000
E = 320000
D = 128

NC = 2
NS = 16
NW = NC * NS
EPW = E // NW
K = 80
NCHUNK = EPW // K
KA = 40
NCA = EPW // KA
RPT = 624
TAIL = N - NS * RPT
RZ = 48

_sc_mesh = plsc.VectorSubcoreMesh(core_axis_name="c", subcore_axis_name="s")
_sc_params = pltpu.CompilerParams(needs_layout_passes=False,
                                  use_tc_tiling_on_sc=False)


@functools.partial(
    pl.kernel,
    out_type=jax.ShapeDtypeStruct((NC * N, 16), jnp.float32),
    mesh=_sc_mesh,
    compiler_params=_sc_params,
    scratch_types=[
        pltpu.VMEM_SHARED((N, 16), jnp.float32),
        pltpu.VMEM((NCHUNK, K), jnp.int32),
        pltpu.VMEM((EPW,), jnp.float32),
        pltpu.VMEM((K, 16), jnp.float32),
        pltpu.VMEM((K, 16), jnp.float32),
        pltpu.VMEM((RZ, 16), jnp.float32),
        pltpu.SemaphoreType.DMA,
        pltpu.SemaphoreType.DMA,
    ],
)
def _deg_kernel(dst_hbm, ew_hbm, out_hbm, acc_sh, didx_v, ew_v, rows_v,
                zbuf16_v, zbuf_v, dsem_a, dsem_b):
    cid = lax.axis_index("c")
    sid = lax.axis_index("s")
    wid = cid * NS + sid

    def zrow(i, carry):
        zbuf_v[i, :] = jnp.zeros((16,), jnp.float32)
        return carry

    lax.fori_loop(0, RZ, zrow, 0)
    for r in range(RPT // RZ):
        pltpu.sync_copy(zbuf_v, acc_sh.at[pl.ds(sid * RPT + r * RZ, RZ)])

    @pl.when(sid == NS - 1)
    def _():
        pltpu.sync_copy(zbuf_v.at[pl.ds(0, TAIL)],
                        acc_sh.at[pl.ds(NS * RPT, TAIL)])

    plsc.subcore_barrier()

    pltpu.sync_copy(dst_hbm.at[wid], didx_v)
    pltpu.sync_copy(ew_hbm.at[wid], ew_v)

    def fill_chunk(rows_v, c):
        def fill(e, carry2):
            w16 = plsc.load_gather(ew_v, [jnp.full((16,), c * K + e, jnp.int32)])
            rows_v[e, :] = w16
            return carry2

        lax.fori_loop(0, K, fill, 0)

    def phase(i, c, rows_v, sem):
        @pl.when(i > 0)
        def _():
            pltpu.make_async_copy(rows_v, acc_sh.at[didx_v.at[c]], sem).wait()

        fill_chunk(rows_v, c)
        pltpu.async_copy(rows_v, acc_sh.at[didx_v.at[c]], sem, add=True)

    def chunk(i, carry):
        phase(i, 2 * i, rows_v, dsem_a)
        phase(i, 2 * i + 1, zbuf16_v, dsem_b)
        return carry

    lax.fori_loop(0, NCHUNK // 2, chunk, 0)
    last = NCHUNK - 1
    pltpu.make_async_copy(rows_v, acc_sh.at[didx_v.at[last]], dsem_a).wait()
    fill_chunk(rows_v, last)
    pltpu.sync_copy(rows_v, acc_sh.at[didx_v.at[last]], add=True)
    pltpu.make_async_copy(zbuf16_v, acc_sh.at[didx_v.at[last]], dsem_b).wait()
    plsc.subcore_barrier()
    pltpu.sync_copy(
        acc_sh.at[pl.ds(sid * RPT, RPT)],
        out_hbm.at[pl.ds(cid * N + sid * RPT, RPT)],
    )

    @pl.when(sid == NS - 1)
    def _():
        pltpu.sync_copy(acc_sh.at[pl.ds(NS * RPT, TAIL)],
                        out_hbm.at[pl.ds(cid * N + NS * RPT, TAIL)])


@functools.partial(
    pl.kernel,
    out_type=jax.ShapeDtypeStruct((E,), jnp.float32),
    mesh=_sc_mesh,
    compiler_params=_sc_params,
    scratch_types=[
        pltpu.VMEM((N,), jnp.float32),
        pltpu.VMEM((EPW,), jnp.int32),
        pltpu.VMEM((EPW,), jnp.float32),
        pltpu.VMEM((EPW,), jnp.float32),
    ],
)
def _edgew_kernel(dinv_hbm, src_hbm, ew_hbm, w2_hbm, dinv_v, src_v, ew_v, w2_v):
    cid = lax.axis_index("c")
    sid = lax.axis_index("s")
    wid = cid * NS + sid
    pltpu.sync_copy(dinv_hbm, dinv_v)
    pltpu.sync_copy(src_hbm.at[wid], src_v)
    pltpu.sync_copy(ew_hbm.at[wid], ew_v)

    def step(j, carry):
        sl = pl.ds(j * 16, 16)
        g = plsc.load_gather(dinv_v, [src_v[sl]])
        w2_v[sl] = ew_v[sl] * g
        return carry

    lax.fori_loop(0, EPW // 16, step, 0)
    pltpu.sync_copy(w2_v, w2_hbm.at[pl.ds(wid * EPW, EPW)])


@functools.partial(
    pl.kernel,
    out_type=jax.ShapeDtypeStruct((NC * N, D), jnp.float32),
    mesh=_sc_mesh,
    compiler_params=_sc_params,
    scratch_types=[
        pltpu.VMEM_SHARED((N, D), jnp.float32),
        pltpu.VMEM((NCA, KA), jnp.int32),
        pltpu.VMEM((NCA, KA), jnp.int32),
        pltpu.VMEM((EPW,), jnp.float32),
        pltpu.VMEM((KA, D), jnp.float32),
        pltpu.VMEM((KA, D), jnp.float32),
        pltpu.VMEM((KA, D), jnp.float32),
        pltpu.SemaphoreType.DMA,
        pltpu.SemaphoreType.DMA,
        pltpu.SemaphoreType.DMA,
        pltpu.SemaphoreType.DMA,
        pltpu.SemaphoreType.DMA,
        pltpu.SemaphoreType.DMA,
    ],
)
def _agg_kernel(h_hbm, src_hbm, dst_hbm, w2_hbm, out_hbm,
                acc_sh, sidx_v, didx_v, w2_v, rows_a, rows_b, rows_c,
                gsem_a, gsem_b, gsem_c, ssem_a, ssem_b, ssem_c):
    cid = lax.axis_index("c")
    sid = lax.axis_index("s")
    wid = cid * NS + sid

    def zrow(i, carry):
        for j in range(D // 16):
            rows_a[i, pl.ds(j * 16, 16)] = jnp.zeros((16,), jnp.float32)
        return carry

    lax.fori_loop(0, KA, zrow, 0)
    for r in range(RPT // KA):
        pltpu.sync_copy(rows_a, acc_sh.at[pl.ds(sid * RPT + r * KA, KA)])
    pltpu.sync_copy(rows_a.at[pl.ds(0, RPT - (RPT // KA) * KA)],
                    acc_sh.at[pl.ds(sid * RPT + (RPT // KA) * KA,
                                    RPT - (RPT // KA) * KA)])

    @pl.when(sid == NS - 1)
    def _():
        pltpu.sync_copy(rows_a.at[pl.ds(0, TAIL)],
                        acc_sh.at[pl.ds(NS * RPT, TAIL)])

    plsc.subcore_barrier()

    pltpu.sync_copy(src_hbm.at[wid], sidx_v)
    pltpu.sync_copy(dst_hbm.at[wid], didx_v)
    pltpu.sync_copy(w2_hbm.at[wid], w2_v)

    def scale(rows_v, c):
        def scale2(i, carry2):
            for u in range(2):
                e = i * 2 + u
                w16 = plsc.load_gather(
                    w2_v, [jnp.full((16,), c * KA + e, jnp.int32)])
                for j in range(D // 16):
                    sl = pl.ds(j * 16, 16)
                    rows_v[e, sl] = rows_v[e, sl] * w16
            return carry2

        lax.fori_loop(0, KA // 2, scale2, 0)

    bufs = (rows_a, rows_b, rows_c)
    gsems = (gsem_a, gsem_b, gsem_c)
    ssems = (ssem_a, ssem_b, ssem_c)
    for b in range(3):
        pltpu.async_copy(h_hbm.at[sidx_v.at[b]], bufs[b], gsems[b])

    def waitg(b, c):
        pltpu.make_async_copy(h_hbm.at[sidx_v.at[c]], bufs[b], gsems[b]).wait()

    def body(i, carry):
        c0 = 3 * i
        cs = (c0, c0 + 1, c0 + 2)
        waitg(0, cs[0])
        scale(rows_a, cs[0])
        pltpu.async_copy(rows_a, acc_sh.at[didx_v.at[cs[0]]], ssem_a, add=True)

        waitg(1, cs[1])
        scale(rows_b, cs[1])
        pltpu.async_copy(rows_b, acc_sh.at[didx_v.at[cs[1]]], ssem_b, add=True)

        pltpu.make_async_copy(rows_a, acc_sh.at[didx_v.at[cs[0]]], ssem_a).wait()
        pltpu.async_copy(h_hbm.at[sidx_v.at[jnp.minimum(c0 + 3, NCA - 1)]],
                         rows_a, gsem_a)

        waitg(2, cs[2])
        scale(rows_c, cs[2])
        pltpu.async_copy(rows_c, acc_sh.at[didx_v.at[cs[2]]], ssem_c, add=True)

        pltpu.make_async_copy(rows_b, acc_sh.at[didx_v.at[cs[1]]], ssem_b).wait()
        pltpu.async_copy(h_hbm.at[sidx_v.at[jnp.minimum(c0 + 4, NCA - 1)]],
                         rows_b, gsem_b)
        pltpu.make_async_copy(rows_c, acc_sh.at[didx_v.at[cs[2]]], ssem_c).wait()
        pltpu.async_copy(h_hbm.at[sidx_v.at[jnp.minimum(c0 + 5, NCA - 1)]],
                         rows_c, gsem_c)
        return carry

    lax.fori_loop(0, (NCA - 1) // 3, body, 0)

    last = NCA - 1
    waitg(0, last)
    scale(rows_a, last)
    pltpu.sync_copy(rows_a, acc_sh.at[didx_v.at[last]], add=True)
    waitg(1, last)
    waitg(2, last)
    plsc.subcore_barrier()
    pltpu.sync_copy(
        acc_sh.at[pl.ds(sid * RPT, RPT)],
        out_hbm.at[pl.ds(cid * N + sid * RPT, RPT)],
    )

    @pl.when(sid == NS - 1)
    def _():
        pltpu.sync_copy(acc_sh.at[pl.ds(NS * RPT, TAIL)],
                        out_hbm.at[pl.ds(cid * N + NS * RPT, TAIL)])


BM = 1000


@functools.partial(
    pl.pallas_call,
    grid=(N // BM,),
    in_specs=[
        pl.BlockSpec((BM, D), lambda i: (i, 0)),
        pl.BlockSpec((D, D), lambda i: (0, 0)),
        pl.BlockSpec((BM, 16), lambda i: (i, 0)),
        pl.BlockSpec((BM, 16), lambda i: (i, 0)),
    ],
    out_specs=[
        pl.BlockSpec((BM, D), lambda i: (i, 0)),
        pl.BlockSpec((BM, 16), lambda i: (i, 0)),
    ],
    out_shape=[
        jax.ShapeDtypeStruct((N, D), jnp.float32),
        jax.ShapeDtypeStruct((N, 16), jnp.float32),
    ],
)
def _mm_dinv_kernel(x_ref, w_ref, d0_ref, d1_ref, h_ref, dinv_ref):
    h_ref[...] = jnp.dot(x_ref[...], w_ref[...],
                         preferred_element_type=jnp.float32)
    deg = d0_ref[...] + d1_ref[...] + 1.0
    dinv_ref[...] = jnp.where(deg > 0, lax.rsqrt(deg), 0.0)


@functools.partial(
    pl.pallas_call,
    grid=(N // BM,),
    in_specs=[
        pl.BlockSpec((BM, D), lambda i: (i, 0)),
        pl.BlockSpec((BM, D), lambda i: (i, 0)),
        pl.BlockSpec((BM, D), lambda i: (i, 0)),
        pl.BlockSpec((BM, 16), lambda i: (i, 0)),
        pl.BlockSpec((1, D), lambda i: (0, 0)),
        pl.BlockSpec((D, D), lambda i: (0, 0)),
    ],
    out_specs=pl.BlockSpec((BM, D), lambda i: (i, 0)),
    out_shape=jax.ShapeDtypeStruct((N, D), jnp.float32),
)
def _combine_mm_kernel(p0_ref, p1_ref, h_ref, dinv_ref, b_ref, w_ref, out_ref):
    d1 = dinv_ref[:, :1]
    y = jnp.maximum(
        d1 * (p0_ref[...] + p1_ref[...]) + (d1 * d1) * h_ref[...] + b_ref[...],
        0.0,
    )
    out_ref[...] = jnp.dot(y, w_ref[...], preferred_element_type=jnp.float32)


@functools.partial(
    pl.pallas_call,
    grid=(N // BM,),
    in_specs=[
        pl.BlockSpec((BM, D), lambda i: (i, 0)),
        pl.BlockSpec((BM, D), lambda i: (i, 0)),
        pl.BlockSpec((BM, D), lambda i: (i, 0)),
        pl.BlockSpec((BM, 16), lambda i: (i, 0)),
        pl.BlockSpec((1, D), lambda i: (0, 0)),
    ],
    out_specs=pl.BlockSpec((BM, D), lambda i: (i, 0)),
    out_shape=jax.ShapeDtypeStruct((N, D), jnp.float32),
)
def _combine_kernel(p0_ref, p1_ref, h_ref, dinv_ref, b_ref, out_ref):
    d1 = dinv_ref[:, :1]
    out_ref[...] = jnp.maximum(
        d1 * (p0_ref[...] + p1_ref[...]) + (d1 * d1) * h_ref[...] + b_ref[...],
        0.0,
    )


def kernel(x, edge_index, edge_attr, W0, b0, W1, b1, W2, b2):
    src = edge_index[0].astype(jnp.int32)
    dst = edge_index[1].astype(jnp.int32)
    src3 = src.reshape(NW, NCA, KA)
    dst3 = dst.reshape(NW, NCA, KA)
    dst3d = dst.reshape(NW, NCHUNK, K)
    src2 = src.reshape(NW, EPW)
    ew2 = edge_attr.astype(jnp.float32).reshape(NW, EPW)

    degp = _deg_kernel(dst3d, ew2)
    h0, dinv16 = _mm_dinv_kernel(x, W0, degp[:N], degp[N:])
    dinv1 = dinv16[:, 0]
    w2 = _edgew_kernel(dinv1, src2, ew2).reshape(NW, EPW)

    b0r = b0.reshape(1, D)
    b1r = b1.reshape(1, D)
    b2r = b2.reshape(1, D)

    P = _agg_kernel(h0, src3, dst3, w2)
    h1 = _combine_mm_kernel(P[:N], P[N:], h0, dinv16, b0r, W1)
    P = _agg_kernel(h1, src3, dst3, w2)
    h2 = _combine_mm_kernel(P[:N], P[N:], h1, dinv16, b1r, W2)
    P = _agg_kernel(h2, src3, dst3, w2)
    return _combine_kernel(P[:N], P[N:], h2, dinv16, b2r)
